# baseline (speedup 1.0000x reference)
import functools

import jax
import jax.numpy as jnp
from jax import lax
from jax.experimental import pallas as pl
from jax.experimental.pallas import tpu as pltpu
from jax.experimental.pallas import tpu_sc as plsc

B = 16384
N_FIELDS = 26
VOCAB = 100000
HIDDEN = 64

_NC = 2
_NS = 16
_NW = _NC * _NS

_ROWS_PER_W = B // _NW
_CB = 8
_IDX_PER_CHUNK = _CB * N_FIELDS
_NCHUNKS = _ROWS_PER_W // _CB
_INV = 1.0 / N_FIELDS


def _body(tab_hbm, idx_hbm, par_hbm, out_hbm,
          idx_v, par_v, buf0, buf1, out_v, sem0, sem1):
    wid = lax.axis_index("s") * _NC + lax.axis_index("c")
    base = wid * _ROWS_PER_W

    pltpu.sync_copy(idx_hbm.at[pl.ds(base * N_FIELDS, _ROWS_PER_W * N_FIELDS)],
                    idx_v)
    pltpu.sync_copy(
        par_hbm.at[pl.ds(base * N_FIELDS, _ROWS_PER_W * N_FIELDS + 16)], par_v)

    def _fire(chunk, buf, sem):
        pltpu.async_copy(
            tab_hbm.at[idx_v.at[pl.ds(chunk * _IDX_PER_CHUNK, _IDX_PER_CHUNK)]],
            buf, sem)

    def _drain(chunk, buf, sem):
        pltpu.make_async_copy(
            tab_hbm.at[idx_v.at[pl.ds(chunk * _IDX_PER_CHUNK, _IDX_PER_CHUNK)]],
            buf, sem).wait()

    def _reduce(chunk, buf):
        def row_step(r, carry):
            row = chunk * _CB + r
            g0 = row * N_FIELDS
            accs = [None] * (HIDDEN // 16)
            for f in range(N_FIELDS):
                g = g0 + f
                pv = pl.multiple_of(par_v[pl.ds(g, 16)][0], 64)
                gl = g - chunk * _IDX_PER_CHUNK
                for c in range(HIDDEN // 16):
                    val = buf[gl, pl.ds(pv + c * 16, 16)]
                    accs[c] = val if f == 0 else accs[c] + val
            for c in range(HIDDEN // 16):
                out_v[row // 2, pl.ds((row % 2) * HIDDEN + c * 16, 16)] = (
                    accs[c] * _INV)
            return carry

        lax.fori_loop(0, _CB, row_step, 0)

    _fire(0, buf0, sem0)
    _fire(1, buf1, sem1)

    def step(g, carry):
        c0 = 2 * g
        c1 = 2 * g + 1
        _drain(c0, buf0, sem0)
        _reduce(c0, buf0)

        @pl.when(g < _NCHUNKS // 2 - 1)
        def _():
            _fire(c0 + 2, buf0, sem0)

        _drain(c1, buf1, sem1)
        _reduce(c1, buf1)

        @pl.when(g < _NCHUNKS // 2 - 1)
        def _():
            _fire(c1 + 2, buf1, sem1)

        return carry

    lax.fori_loop(0, _NCHUNKS // 2, step, 0)

    obase = pl.multiple_of(wid * (_ROWS_PER_W // 2), 256)
    pltpu.sync_copy(out_v, out_hbm.at[pl.ds(obase, _ROWS_PER_W // 2)])


@jax.jit
def _run(tab_pairs, idx_flat, par_flat):
    mesh = plsc.VectorSubcoreMesh(core_axis_name="c", subcore_axis_name="s")
    return pl.kernel(
        _body,
        mesh=mesh,
        out_type=jax.ShapeDtypeStruct((B // 2, 2 * HIDDEN), jnp.float32),
        scratch_types=[
            pltpu.VMEM((_ROWS_PER_W * N_FIELDS,), jnp.int32),
            pltpu.VMEM((_ROWS_PER_W * N_FIELDS + 16,), jnp.int32),
            pltpu.VMEM((_IDX_PER_CHUNK, 128), jnp.float32),
            pltpu.VMEM((_IDX_PER_CHUNK, 128), jnp.float32),
            pltpu.VMEM((_ROWS_PER_W // 2, 2 * HIDDEN), jnp.float32),
            pltpu.SemaphoreType.DMA,
            pltpu.SemaphoreType.DMA,
        ],
    )(tab_pairs, idx_flat, par_flat)


def kernel(x, tables):
    flat = (x.astype(jnp.int32)
            + (jnp.arange(N_FIELDS, dtype=jnp.int32) * VOCAB)[None, :]
            ).reshape(-1)
    idx_flat = flat >> 1
    par_flat = jnp.pad((flat & 1) << 6, (0, 16))
    tab_pairs = tables.reshape(N_FIELDS * VOCAB // 2, 2 * HIDDEN)
    return _run(tab_pairs, idx_flat, par_flat).reshape(B, HIDDEN)


# speedup vs baseline: 1.0381x; 1.0381x over previous
import functools

import jax
import jax.numpy as jnp
from jax import lax
from jax.experimental import pallas as pl
from jax.experimental.pallas import tpu as pltpu
from jax.experimental.pallas import tpu_sc as plsc

B = 16384
N_FIELDS = 26
VOCAB = 100000
HIDDEN = 64

_NC = 2
_NS = 16
_NW = _NC * _NS

_ROWS_PER_W = B // _NW
_CB = 16
_IDX_PER_CHUNK = _CB * N_FIELDS
_NCHUNKS = _ROWS_PER_W // _CB
_INV = 1.0 / N_FIELDS


def _body(tab_hbm, idx_hbm, out_hbm, idx_v, buf0, buf1, out_v, sem0, sem1):
    wid = lax.axis_index("s") * _NC + lax.axis_index("c")
    base = wid * _ROWS_PER_W

    pltpu.sync_copy(idx_hbm.at[pl.ds(base * N_FIELDS, _ROWS_PER_W * N_FIELDS)],
                    idx_v)

    def _fire(chunk, buf, sem):
        pltpu.async_copy(
            tab_hbm.at[idx_v.at[pl.ds(chunk * _IDX_PER_CHUNK, _IDX_PER_CHUNK)]],
            buf, sem)

    def _drain(chunk, buf, sem):
        pltpu.make_async_copy(
            tab_hbm.at[idx_v.at[pl.ds(chunk * _IDX_PER_CHUNK, _IDX_PER_CHUNK)]],
            buf, sem).wait()

    def _reduce(chunk, buf):
        def row_step(r, carry):
            row = chunk * _CB + r
            accs = [None] * (HIDDEN // 16)
            for f in range(N_FIELDS):
                gl = r * N_FIELDS + f
                for c in range(HIDDEN // 16):
                    val = buf[gl, pl.ds(c * 16, 16)]
                    accs[c] = val if f == 0 else accs[c] + val
            for c in range(HIDDEN // 16):
                out_v[row, pl.ds(c * 16, 16)] = accs[c] * _INV
            return carry

        lax.fori_loop(0, _CB, row_step, 0)

    _fire(0, buf0, sem0)
    _fire(1, buf1, sem1)

    def step(g, carry):
        c0 = 2 * g
        c1 = 2 * g + 1
        _drain(c0, buf0, sem0)
        _reduce(c0, buf0)

        @pl.when(g < _NCHUNKS // 2 - 1)
        def _():
            _fire(c0 + 2, buf0, sem0)

        _drain(c1, buf1, sem1)
        _reduce(c1, buf1)

        @pl.when(g < _NCHUNKS // 2 - 1)
        def _():
            _fire(c1 + 2, buf1, sem1)

        return carry

    lax.fori_loop(0, _NCHUNKS // 2, step, 0)

    pltpu.sync_copy(out_v, out_hbm.at[pl.ds(base, _ROWS_PER_W)])


@jax.jit
def _run(tab_flat, idx_flat):
    mesh = plsc.VectorSubcoreMesh(core_axis_name="c", subcore_axis_name="s")
    return pl.kernel(
        _body,
        mesh=mesh,
        out_type=jax.ShapeDtypeStruct((B, HIDDEN), jnp.float32),
        scratch_types=[
            pltpu.VMEM((_ROWS_PER_W * N_FIELDS,), jnp.int32),
            pltpu.VMEM((_IDX_PER_CHUNK, HIDDEN), jnp.float32),
            pltpu.VMEM((_IDX_PER_CHUNK, HIDDEN), jnp.float32),
            pltpu.VMEM((_ROWS_PER_W, HIDDEN), jnp.float32),
            pltpu.SemaphoreType.DMA,
            pltpu.SemaphoreType.DMA,
        ],
        compiler_params=pltpu.CompilerParams(use_tc_tiling_on_sc=False),
    )(tab_flat, idx_flat)


def kernel(x, tables):
    idx_flat = (x.astype(jnp.int32)
                + (jnp.arange(N_FIELDS, dtype=jnp.int32) * VOCAB)[None, :]
                ).reshape(-1)
    tab_flat = tables.reshape(N_FIELDS * VOCAB, HIDDEN)
    return _run(tab_flat, idx_flat)
